# Initial kernel scaffold; baseline (speedup 1.0000x reference)
import jax
import jax.numpy as jnp
from jax import lax
from jax.experimental import pallas as pl
from jax.experimental.pallas import tpu as pltpu
from jax.experimental.pallas import tpu_sc as plsc

EMB = 64
SCALE = 8.0
B, L = 4096, 200
TOK = B * L
NC, NS = 2, 16
NW = NC * NS
TPW = TOK // NW
RPG = 4
GSZ = RPG * 128
NG = TPW // GSZ
TROWS = TOK // 128
WROWS = TPW // 128


def _scale_body(x_ref, o_ref):
    o_ref[...] = x_ref[...] * SCALE


def _scale_table(table):
    t2 = table.reshape(50000, 128)
    out = pl.pallas_call(
        _scale_body,
        out_shape=jax.ShapeDtypeStruct((50000, 128), jnp.float32),
        grid=(25,),
        in_specs=[pl.BlockSpec((2000, 128), lambda i: (i, 0))],
        out_specs=pl.BlockSpec((2000, 128), lambda i: (i, 0)),
    )(t2)
    return out.reshape(100000, EMB)


def _emb_body(table, toks, out, idx_v, rows_v, si0, si1, sg0, sg1, so0, so1):
    wid = lax.axis_index("s") * NC + lax.axis_index("c")
    row0 = wid * WROWS
    s_idx = (si0, si1)
    s_g = (sg0, sg1)
    s_o = (so0, so1)

    def fire_idx(g, s):
        pltpu.async_copy(toks.at[pl.ds(row0 + g * RPG, RPG)],
                         idx_v.at[pl.ds(s * RPG, RPG)], s_idx[s])

    def wait_idx(g, s):
        pltpu.make_async_copy(toks.at[pl.ds(row0 + g * RPG, RPG)],
                              idx_v.at[pl.ds(s * RPG, RPG)], s_idx[s]).wait()

    def run_gathers(s):
        descs = [
            pltpu.async_copy(table.at[idx_v.at[s * RPG + j]],
                             rows_v.at[s * RPG + j], s_g[s])
            for j in range(RPG)
        ]
        for d in descs:
            d.wait()

    def fire_out(g, s):
        pltpu.async_copy(rows_v.at[pl.ds(s * RPG, RPG)],
                         out.at[pl.ds(row0 + g * RPG, RPG)], s_o[s])

    def wait_out(g, s):
        pltpu.make_async_copy(rows_v.at[pl.ds(s * RPG, RPG)],
                              out.at[pl.ds(row0 + g * RPG, RPG)], s_o[s]).wait()

    fire_idx(0, 0)
    fire_idx(1, 1)
    for g in (0, 1):
        s = g
        wait_idx(g, s)
        run_gathers(s)
        fire_out(g, s)
        fire_idx(g + 2, s)

    @pl.loop(0, (NG - 4) // 2)
    def _steady(i):
        for s in range(2):
            g = 2 + i * 2 + s
            wait_idx(g, s)
            wait_out(g - 2, s)
            run_gathers(s)
            fire_out(g, s)
            fire_idx(g + 2, s)

    for g in (NG - 2, NG - 1):
        s = g % 2
        wait_idx(g, s)
        wait_out(g - 2, s)
        run_gathers(s)
        fire_out(g, s)
    for g in (NG - 2, NG - 1):
        wait_out(g, g % 2)


def _emb_lookup(table, tok2d):
    mesh = plsc.VectorSubcoreMesh(core_axis_name="c", subcore_axis_name="s",
                                  num_cores=NC, num_subcores=NS)
    f = pl.kernel(
        _emb_body,
        out_type=jax.ShapeDtypeStruct((TROWS, 128, EMB), jnp.float32),
        mesh=mesh,
        scratch_types=[
            pltpu.VMEM((2 * RPG, 128), jnp.int32),
            pltpu.VMEM((2 * RPG, 128, EMB), jnp.float32),
            pltpu.SemaphoreType.DMA,
            pltpu.SemaphoreType.DMA,
            pltpu.SemaphoreType.DMA,
            pltpu.SemaphoreType.DMA,
            pltpu.SemaphoreType.DMA,
            pltpu.SemaphoreType.DMA,
        ],
    )
    return f(table, tok2d)


def kernel(tokens, table):
    scaled = _scale_table(table)
    tok2d = tokens.reshape(TROWS, 128)
    out = _emb_lookup(scaled, tok2d)
    return out.reshape(B, L, EMB)


# speedup vs baseline: 4.0960x; 4.0960x over previous
import jax
import jax.numpy as jnp
from jax import lax
from jax.experimental import pallas as pl
from jax.experimental.pallas import tpu as pltpu
from jax.experimental.pallas import tpu_sc as plsc

EMB = 64
SCALE = 8.0
B, L = 4096, 200
TOK = B * L
NC, NS = 2, 16
NW = NC * NS
TPW = TOK // NW
RPG = 4
GSZ = RPG * 128
NG = TPW // GSZ
TROWS = TOK // 128
WROWS = TPW // 128


def _scale_body(x_ref, o_ref):
    o_ref[...] = x_ref[...] * SCALE


def _scale_table(table):
    t2 = table.reshape(50000, 128)
    out = pl.pallas_call(
        _scale_body,
        out_shape=jax.ShapeDtypeStruct((50000, 128), jnp.float32),
        grid=(25,),
        in_specs=[pl.BlockSpec((2000, 128), lambda i: (i, 0))],
        out_specs=pl.BlockSpec((2000, 128), lambda i: (i, 0)),
    )(t2)
    return out.reshape(100000, EMB)


def _emb_body(table, toks, out, idx_v, rows_v, si0, si1, sg0, sg1, so0, so1):
    wid = lax.axis_index("s") * NC + lax.axis_index("c")
    row0 = wid * WROWS
    s_idx = (si0, si1)
    s_g = (sg0, sg1)
    s_o = (so0, so1)

    def fire_idx(g, s):
        pltpu.async_copy(toks.at[pl.ds(row0 + g * RPG, RPG)],
                         idx_v.at[pl.ds(s * RPG, RPG)], s_idx[s])

    def wait_idx(g, s):
        pltpu.make_async_copy(toks.at[pl.ds(row0 + g * RPG, RPG)],
                              idx_v.at[pl.ds(s * RPG, RPG)], s_idx[s]).wait()

    def run_gathers(s):
        descs = [
            pltpu.async_copy(table.at[idx_v.at[s * RPG + j]],
                             rows_v.at[s * RPG + j], s_g[s])
            for j in range(RPG)
        ]
        for d in descs:
            d.wait()

    def fire_out(g, s):
        pltpu.async_copy(rows_v.at[pl.ds(s * RPG, RPG)],
                         out.at[pl.ds(row0 + g * RPG, RPG)], s_o[s])

    def wait_out(g, s):
        pltpu.make_async_copy(rows_v.at[pl.ds(s * RPG, RPG)],
                              out.at[pl.ds(row0 + g * RPG, RPG)], s_o[s]).wait()

    fire_idx(0, 0)
    fire_idx(1, 1)
    for g in (0, 1):
        s = g
        wait_idx(g, s)
        run_gathers(s)
        fire_out(g, s)
        fire_idx(g + 2, s)

    @pl.loop(0, (NG - 4) // 2)
    def _steady(i):
        for s in range(2):
            g = 2 + i * 2 + s
            wait_idx(g, s)
            wait_out(g - 2, s)
            run_gathers(s)
            fire_out(g, s)
            fire_idx(g + 2, s)

    for g in (NG - 2, NG - 1):
        s = g % 2
        wait_idx(g, s)
        wait_out(g - 2, s)
        run_gathers(s)
        fire_out(g, s)
    for g in (NG - 2, NG - 1):
        wait_out(g, g % 2)


def _emb_lookup(table, tok2d):
    mesh = plsc.VectorSubcoreMesh(core_axis_name="c", subcore_axis_name="s",
                                  num_cores=NC, num_subcores=NS)
    f = pl.kernel(
        _emb_body,
        out_type=jax.ShapeDtypeStruct((TROWS, 128, EMB), jnp.float32),
        mesh=mesh,
        scratch_types=[
            pltpu.VMEM((2 * RPG, 128), jnp.int32),
            pltpu.VMEM((2 * RPG, 128, EMB), jnp.float32),
            pltpu.SemaphoreType.DMA,
            pltpu.SemaphoreType.DMA,
            pltpu.SemaphoreType.DMA,
            pltpu.SemaphoreType.DMA,
            pltpu.SemaphoreType.DMA,
            pltpu.SemaphoreType.DMA,
        ],
        compiler_params=pltpu.CompilerParams(use_tc_tiling_on_sc=False),
    )
    return f(table, tok2d)


def kernel(tokens, table):
    scaled = _scale_table(table)
    tok2d = tokens.reshape(TROWS, 128)
    out = _emb_lookup(scaled, tok2d)
    return out.reshape(B, L, EMB)
